# baseline (device time: 69913 ns/iter reference)
import jax
import jax.numpy as jnp
from jax import lax
from jax.experimental import pallas as pl
from jax.experimental.pallas import tpu as pltpu

N_DEV = 8

_COL_PARTS = (
    (0, 768, (1, 3, 4)),
    (768, 640, (3, 4, 1)),
    (1408, 640, (4, 1, 3)),
)
PARTS = tuple(
    dict(c0=c0, nc=nc, order=order, r0=r0)
    for c0, nc, order in _COL_PARTS
    for r0 in (0, 128)
)

_RS_OFF = (0, 4, 6)
_AG_OFF = (0, 1, 3)


def _span(masks):
    s = {0}
    for m in masks:
        s |= {x ^ m for x in s}
    return sorted(s)


def _rs_sched(order):
    sendks = [None, None, [order[2]]]
    for j in (1, 0):
        msk = order[j]
        full = {msk ^ s for s in _span(order[j + 1:])}
        pri = [msk ^ kk for kk in sendks[j + 1]]
        sendks[j] = pri + sorted(full - set(pri))
    return [
        (order[j], sendks[j], len(sendks[j + 1]) if j < 2 else 1)
        for j in range(3)
    ]


def _ag_sched(order):
    lseq = (order[2], order[1], order[0])
    held = [0]
    waves = []
    for a in range(3):
        rk = [lseq[a] ^ kk for kk in held[:2 ** a]]
        waves.append(rk)
        held += rk
    return lseq, held, waves


_RS = tuple(_rs_sched(P["order"]) for P in PARTS)
_AG = tuple(_ag_sched(P["order"]) for P in PARTS)


def kernel(x, w_mat):
    m, k = x.shape
    _, n = w_mat.shape
    chunk = m // N_DEV
    rh = chunk // 2

    def body(x_ref, w_ref, out_ref, *scr):
        me = lax.axis_index("i")

        barrier_sem = pltpu.get_barrier_semaphore()
        for msk in (1, 3, 4):
            pl.semaphore_signal(barrier_sem, inc=1, device_id=(me ^ msk,),
                                device_id_type=pl.DeviceIdType.MESH)
        pl.semaphore_wait(barrier_sem, 3)

        def prow(p, c):
            return pl.ds(c * chunk + PARTS[p]["r0"], rh)

        def P(p):
            return scr[p * 3:(p + 1) * 3]

        all_rdmas = []
        rs_desc = {}
        ag_desc = {}
        ag_ctr = [7] * len(PARTS)

        def rs_send_one(p, j, t):
            rs_rcv, ssems, rsems = P(p)
            msk, sendk, _ = _RS[p][j]
            c0, nc = PARTS[p]["c0"], PARTS[p]["nc"]
            slot = _RS_OFF[j] + t
            r = pltpu.make_async_remote_copy(
                src_ref=out_ref.at[prow(p, me ^ sendk[t]), pl.ds(c0, nc)],
                dst_ref=rs_rcv.at[pl.ds(slot * rh, rh)],
                send_sem=ssems.at[slot],
                recv_sem=rsems.at[slot],
                device_id=(me ^ msk,),
                device_id_type=pl.DeviceIdType.MESH,
            )
            r.start()
            all_rdmas.append(r)
            rs_desc.setdefault((p, j), []).append(r)

        def ag_send(p, a, h):
            _, ssems, rsems = P(p)
            lseq, held, _ = _AG[p]
            c0, nc = PARTS[p]["c0"], PARTS[p]["nc"]
            r = pltpu.make_async_remote_copy(
                src_ref=out_ref.at[prow(p, me ^ held[h]), pl.ds(c0, nc)],
                dst_ref=out_ref.at[prow(p, me ^ held[h]), pl.ds(c0, nc)],
                send_sem=ssems.at[ag_ctr[p]],
                recv_sem=rsems.at[7 + _AG_OFF[a] + h],
                device_id=(me ^ lseq[a],),
                device_id_type=pl.DeviceIdType.MESH,
            )
            ag_ctr[p] += 1
            r.start()
            all_rdmas.append(r)
            ag_desc[(p, a, h)] = r

        def rs_recv(p, j, t):
            msk, sendk, _ = _RS[p][j]
            c0, nc = PARTS[p]["c0"], PARTS[p]["nc"]
            rs_rcv = P(p)[0]
            rs_desc[(p, j)][t].wait_recv()
            slot = _RS_OFF[j] + t
            out_ref[prow(p, me ^ msk ^ sendk[t]), pl.ds(c0, nc)] += rs_rcv[
                pl.ds(slot * rh, rh), :]

        for i, (c0, nc, _) in enumerate(_COL_PARTS):
            out_ref[:, c0:c0 + nc] = jnp.dot(
                x_ref[...].astype(jnp.bfloat16),
                w_ref[:, c0:c0 + nc].astype(jnp.bfloat16),
                preferred_element_type=jnp.float32,
            ).astype(jnp.bfloat16)
            for t in range(len(_RS[2 * i][0][1])):
                for p in (2 * i, 2 * i + 1):
                    rs_send_one(p, 0, t)

        npart = len(PARTS)
        for j in range(2):
            npri = _RS[0][j][2]
            nall = len(_RS[0][j][1])
            for t in range(npri):
                for p in range(npart):
                    rs_recv(p, j, t)
                    rs_send_one(p, j + 1, t)
            for t in range(npri, nall):
                for p in range(npart):
                    rs_recv(p, j, t)
        for p in range(npart):
            rs_recv(p, 2, 0)
            c0, nc = PARTS[p]["c0"], PARTS[p]["nc"]
            out_ref[prow(p, me), pl.ds(c0, nc)] = jnp.maximum(
                out_ref[prow(p, me), pl.ds(c0, nc)], 0.0)
            for a in range(3):
                ag_send(p, a, 0)

        for a in range(3):
            for t in range(2 ** a):
                for p in range(npart):
                    ag_desc[(p, a, t)].wait_recv()
                    hr = 2 ** a + t
                    for a2 in range(a + 1, 3):
                        if hr < 2 ** a2:
                            ag_send(p, a2, hr)

        for r in all_rdmas:
            r.wait_send()

    scratch = []
    for prt in PARTS:
        scratch += [
            pltpu.VMEM((7 * rh, prt["nc"]), jnp.bfloat16),
            pltpu.SemaphoreType.DMA((14,)),
            pltpu.SemaphoreType.DMA((14,)),
        ]

    return pl.pallas_call(
        body,
        out_shape=jax.ShapeDtypeStruct((m, n), jnp.bfloat16),
        in_specs=[
            pl.BlockSpec(memory_space=pltpu.VMEM),
            pl.BlockSpec(memory_space=pltpu.VMEM),
        ],
        out_specs=pl.BlockSpec(memory_space=pltpu.VMEM),
        scratch_shapes=scratch,
        compiler_params=pltpu.CompilerParams(collective_id=0),
    )(x, w_mat)


# device time: 40492 ns/iter; 1.7266x vs baseline; 1.7266x over previous
import jax
import jax.numpy as jnp
from jax import lax
from jax.experimental import pallas as pl
from jax.experimental.pallas import tpu as pltpu

N_DEV = 8

_COL_PARTS = (
    (0, 768, (1, 3, 4)),
    (768, 640, (3, 4, 1)),
    (1408, 640, (4, 1, 3)),
)
PARTS = tuple(
    dict(c0=c0, nc=nc, order=order, r0=r0)
    for c0, nc, order in _COL_PARTS
    for r0 in (0, 128)
)

_RS_OFF = (0, 4, 6)
_AG_OFF = (0, 1, 3)


def _span(masks):
    s = {0}
    for m in masks:
        s |= {x ^ m for x in s}
    return sorted(s)


def _rs_sched(order):
    sendks = [None, None, [order[2]]]
    for j in (1, 0):
        msk = order[j]
        full = {msk ^ s for s in _span(order[j + 1:])}
        pri = [msk ^ kk for kk in sendks[j + 1]]
        sendks[j] = pri + sorted(full - set(pri))
    return [
        (order[j], sendks[j], len(sendks[j + 1]) if j < 2 else 1)
        for j in range(3)
    ]


def _ag_sched(order):
    lseq = (order[2], order[1], order[0])
    held = [0]
    waves = []
    for a in range(3):
        rk = [lseq[a] ^ kk for kk in held[:2 ** a]]
        waves.append(rk)
        held += rk
    return lseq, held, waves


_RS = tuple(_rs_sched(P["order"]) for P in PARTS)
_AG = tuple(_ag_sched(P["order"]) for P in PARTS)


def kernel(x, w_mat):
    m, k = x.shape
    _, n = w_mat.shape
    chunk = m // N_DEV
    rh = chunk // 2

    def body(x_ref, w_ref, out_ref, *scr):
        me = lax.axis_index("i")

        barrier_sem = pltpu.get_barrier_semaphore()
        for msk in (1, 3, 4):
            pl.semaphore_signal(barrier_sem, inc=1, device_id=(me ^ msk,),
                                device_id_type=pl.DeviceIdType.MESH)
        pl.semaphore_wait(barrier_sem, 3)

        def prow(p, c):
            return pl.ds(c * chunk + PARTS[p]["r0"], rh)

        def P(p):
            return scr[p * 3:(p + 1) * 3]

        all_rdmas = []
        rs_desc = {}
        ag_desc = {}
        ag_ctr = [7] * len(PARTS)

        def rs_send_one(p, j, t):
            rs_rcv, ssems, rsems = P(p)
            msk, sendk, _ = _RS[p][j]
            c0, nc = PARTS[p]["c0"], PARTS[p]["nc"]
            slot = _RS_OFF[j] + t
            r = pltpu.make_async_remote_copy(
                src_ref=out_ref.at[prow(p, me ^ sendk[t]), pl.ds(c0, nc)],
                dst_ref=rs_rcv.at[pl.ds(slot * rh, rh)],
                send_sem=ssems.at[slot],
                recv_sem=rsems.at[slot],
                device_id=(me ^ msk,),
                device_id_type=pl.DeviceIdType.MESH,
            )
            r.start()
            all_rdmas.append(r)
            rs_desc.setdefault((p, j), []).append(r)

        def ag_send(p, a, h):
            _, ssems, rsems = P(p)
            lseq, held, _ = _AG[p]
            c0, nc = PARTS[p]["c0"], PARTS[p]["nc"]
            r = pltpu.make_async_remote_copy(
                src_ref=out_ref.at[prow(p, me ^ held[h]), pl.ds(c0, nc)],
                dst_ref=out_ref.at[prow(p, me ^ held[h]), pl.ds(c0, nc)],
                send_sem=ssems.at[ag_ctr[p]],
                recv_sem=rsems.at[7 + _AG_OFF[a] + h],
                device_id=(me ^ lseq[a],),
                device_id_type=pl.DeviceIdType.MESH,
            )
            ag_ctr[p] += 1
            r.start()
            all_rdmas.append(r)
            ag_desc[(p, a, h)] = r

        def rs_recv(p, j, t):
            msk, sendk, _ = _RS[p][j]
            c0, nc = PARTS[p]["c0"], PARTS[p]["nc"]
            rs_rcv = P(p)[0]
            rs_desc[(p, j)][t].wait_recv()
            slot = _RS_OFF[j] + t
            out_ref[prow(p, me ^ msk ^ sendk[t]), pl.ds(c0, nc)] += rs_rcv[
                pl.ds(slot * rh, rh), :]

        def gemm_chunk(c, c0, nc):
            out_ref[pl.ds(c * chunk, chunk), pl.ds(c0, nc)] = jnp.dot(
                x_ref[pl.ds(c * chunk, chunk), :].astype(jnp.bfloat16),
                w_ref[:, c0:c0 + nc].astype(jnp.bfloat16),
                preferred_element_type=jnp.float32,
            ).astype(jnp.bfloat16)

        for i, (c0, nc, order) in enumerate(_COL_PARTS):
            for t, kk in enumerate(_RS[2 * i][0][1]):
                gemm_chunk(me ^ kk, c0, nc)
                for p in (2 * i, 2 * i + 1):
                    rs_send_one(p, 0, t)
        for i, (c0, nc, order) in enumerate(_COL_PARTS):
            for s in _span(order[1:]):
                gemm_chunk(me ^ s, c0, nc)

        npart = len(PARTS)
        for j in range(2):
            npri = _RS[0][j][2]
            nall = len(_RS[0][j][1])
            for t in range(npri):
                for p in range(npart):
                    rs_recv(p, j, t)
                    rs_send_one(p, j + 1, t)
            for t in range(npri, nall):
                for p in range(npart):
                    rs_recv(p, j, t)
        for p in range(npart):
            rs_recv(p, 2, 0)
            c0, nc = PARTS[p]["c0"], PARTS[p]["nc"]
            out_ref[prow(p, me), pl.ds(c0, nc)] = jnp.maximum(
                out_ref[prow(p, me), pl.ds(c0, nc)], 0.0)
            for a in range(3):
                ag_send(p, a, 0)

        for a in range(3):
            for t in range(2 ** a):
                for p in range(npart):
                    ag_desc[(p, a, t)].wait_recv()
                    hr = 2 ** a + t
                    for a2 in range(a + 1, 3):
                        if hr < 2 ** a2:
                            ag_send(p, a2, hr)

        for r in all_rdmas:
            r.wait_send()

    scratch = []
    for prt in PARTS:
        scratch += [
            pltpu.VMEM((7 * rh, prt["nc"]), jnp.bfloat16),
            pltpu.SemaphoreType.DMA((14,)),
            pltpu.SemaphoreType.DMA((14,)),
        ]

    return pl.pallas_call(
        body,
        out_shape=jax.ShapeDtypeStruct((m, n), jnp.bfloat16),
        in_specs=[
            pl.BlockSpec(memory_space=pltpu.VMEM),
            pl.BlockSpec(memory_space=pltpu.VMEM),
        ],
        out_specs=pl.BlockSpec(memory_space=pltpu.VMEM),
        scratch_shapes=scratch,
        compiler_params=pltpu.CompilerParams(collective_id=0),
    )(x, w_mat)
